# Initial kernel scaffold; baseline (speedup 1.0000x reference)
import jax
import jax.numpy as jnp
from jax.experimental import pallas as pl


def kernel(W):
    raise NotImplementedError("write your pallas kernel here")


# speedup vs baseline: 18.1941x; 18.1941x over previous
import jax
import jax.numpy as jnp
from jax.experimental import pallas as pl

_N = 4096
_R = 256
_K = 10


def _sym_block(wr, wc, i):
    r, n = wr.shape
    s = jnp.maximum(jnp.maximum(wr, 0.0), jnp.maximum(wc, 0.0).T)
    col = jax.lax.broadcasted_iota(jnp.int32, (r, n), 1)
    row = jax.lax.broadcasted_iota(jnp.int32, (r, n), 0) + i * r
    return jnp.where(col == row, s + 1.0, s)


def _phase1(wr_ref, wc_ref, t_ref, s_ref):
    i = pl.program_id(0)
    s = _sym_block(wr_ref[...], wc_ref[...], i)
    r = s.shape[0]
    t = jnp.full((r, 1), jnp.inf, jnp.float32)
    c = jnp.zeros((r, 1), jnp.float32)
    for _ in range(_K):
        masked = jnp.where(s < t, s, -1.0)
        m = jnp.max(masked, axis=1, keepdims=True)
        cnt = jnp.sum(jnp.where(masked == m, 1.0, 0.0), axis=1, keepdims=True)
        upd = c < float(_K)
        t = jnp.where(upd, m, t)
        c = jnp.where(upd, c + cnt, c)
    ssum = jnp.sum(jnp.where(s >= t, s, 0.0), axis=1, keepdims=True)
    t_ref[0, 0, :] = t[:, 0]
    s_ref[0, 0, :] = ssum[:, 0]


def _phase2(wr_ref, wc_ref, t_ref, s_ref, o_ref):
    i = pl.program_id(0)
    s = _sym_block(wr_ref[...], wc_ref[...], i)
    t = t_ref[0, 0, :][:, None]
    recip = (1.0 / (s_ref[0, 0, :] + 1e-8))[:, None]
    o_ref[...] = jnp.where(s >= t, s * recip, 0.0)


def kernel(W):
    n = W.shape[0]
    g = n // _R
    row_spec = pl.BlockSpec((_R, n), lambda i: (i, 0))
    col_spec = pl.BlockSpec((n, _R), lambda i: (0, i))
    vec_spec = pl.BlockSpec((1, 1, _R), lambda i: (i, 0, 0))
    vec_shape = jax.ShapeDtypeStruct((g, 1, _R), jnp.float32)

    t, ssum = pl.pallas_call(
        _phase1,
        grid=(g,),
        in_specs=[row_spec, col_spec],
        out_specs=[vec_spec, vec_spec],
        out_shape=[vec_shape, vec_shape],
    )(W, W)

    out = pl.pallas_call(
        _phase2,
        grid=(g,),
        in_specs=[row_spec, col_spec, vec_spec, vec_spec],
        out_specs=pl.BlockSpec((_R, n), lambda i: (i, 0)),
        out_shape=jax.ShapeDtypeStruct((n, n), jnp.float32),
    )(W, W, t, ssum)
    return out


# speedup vs baseline: 24.9784x; 1.3729x over previous
import jax
import jax.numpy as jnp
from jax.experimental import pallas as pl

_N = 4096
_R = 256
_K = 10


def _sym_block(wr, wc, i):
    r, n = wr.shape
    s = jnp.maximum(jnp.maximum(wr, 0.0), jnp.maximum(wc, 0.0).T)
    col = jax.lax.broadcasted_iota(jnp.int32, (r, n), 1)
    row = jax.lax.broadcasted_iota(jnp.int32, (r, n), 0) + i * r
    return jnp.where(col == row, s + 1.0, s)


def _bitonic_desc(arrs):
    a = list(arrs)
    n = len(a)
    k = 2
    while k <= n:
        j = k // 2
        while j >= 1:
            for i in range(n):
                l = i ^ j
                if l > i:
                    desc = (i & k) == 0
                    hi = jnp.maximum(a[i], a[l])
                    lo = jnp.minimum(a[i], a[l])
                    a[i], a[l] = (hi, lo) if desc else (lo, hi)
            j //= 2
        k *= 2
    return a


def _threshold_and_sum(s):
    r, n = s.shape
    nchunks = n // 128
    if nchunks > _K and (nchunks & (nchunks - 1)) == 0:
        chunks = [s[:, g * 128:(g + 1) * 128] for g in range(nchunks)]
        cands = jnp.concatenate(_bitonic_desc(chunks)[:_K], axis=1)
    else:
        cands = s
    t = jnp.full((r, 1), jnp.inf, jnp.float32)
    c = jnp.zeros((r, 1), jnp.float32)
    for _ in range(_K):
        masked = jnp.where(cands < t, cands, -1.0)
        m = jnp.max(masked, axis=1, keepdims=True)
        cnt = jnp.sum(jnp.where(cands >= m, 1.0, 0.0), axis=1, keepdims=True)
        upd = c < float(_K)
        t = jnp.where(upd, m, t)
        c = jnp.where(upd, cnt, c)
    ssum = jnp.sum(jnp.where(s >= t, s, 0.0), axis=1, keepdims=True)
    return t, ssum


def _phase1(wr_ref, wc_ref, t_ref, s_ref):
    i = pl.program_id(0)
    s = _sym_block(wr_ref[...], wc_ref[...], i)
    t, ssum = _threshold_and_sum(s)
    t_ref[0, 0, :] = t[:, 0]
    s_ref[0, 0, :] = ssum[:, 0]


def _phase2(wr_ref, wc_ref, t_ref, s_ref, o_ref):
    i = pl.program_id(0)
    s = _sym_block(wr_ref[...], wc_ref[...], i)
    t = t_ref[0, 0, :][:, None]
    recip = (1.0 / (s_ref[0, 0, :] + 1e-8))[:, None]
    o_ref[...] = jnp.where(s >= t, s * recip, 0.0)


def kernel(W):
    n = W.shape[0]
    g = n // _R
    row_spec = pl.BlockSpec((_R, n), lambda i: (i, 0))
    col_spec = pl.BlockSpec((n, _R), lambda i: (0, i))
    vec_spec = pl.BlockSpec((1, 1, _R), lambda i: (i, 0, 0))
    vec_shape = jax.ShapeDtypeStruct((g, 1, _R), jnp.float32)

    t, ssum = pl.pallas_call(
        _phase1,
        grid=(g,),
        in_specs=[row_spec, col_spec],
        out_specs=[vec_spec, vec_spec],
        out_shape=[vec_shape, vec_shape],
    )(W, W)

    out = pl.pallas_call(
        _phase2,
        grid=(g,),
        in_specs=[row_spec, col_spec, vec_spec, vec_spec],
        out_specs=pl.BlockSpec((_R, n), lambda i: (i, 0)),
        out_shape=jax.ShapeDtypeStruct((n, n), jnp.float32),
    )(W, W, t, ssum)
    return out


# speedup vs baseline: 29.9901x; 1.2006x over previous
import jax
import jax.numpy as jnp
from jax.experimental import pallas as pl

_N = 4096
_R = 256
_K = 10


def _bitonic_topk_plan(n, k):
    ex = []
    kk = 2
    while kk <= n:
        j = kk // 2
        while j >= 1:
            for i in range(n):
                l = i ^ j
                if l > i:
                    ex.append((i, l, (i & kk) == 0))
            j //= 2
        kk *= 2
    needed = set(range(k))
    plan = []
    for (i, l, d) in reversed(ex):
        ni, nl = i in needed, l in needed
        if not (ni or nl):
            continue
        plan.append((i, l, d, ni, nl))
        needed.add(i)
        needed.add(l)
    plan.reverse()
    return plan


_PLAN32 = _bitonic_topk_plan(32, _K)


def _lane_topk(chunks, plan):
    a = list(chunks)
    for (i, l, desc, ni, nl) in plan:
        x, y = a[i], a[l]
        hi = jnp.maximum(x, y) if (ni if desc else nl) else None
        lo = jnp.minimum(x, y) if (nl if desc else ni) else None
        if desc:
            if ni:
                a[i] = hi
            if nl:
                a[l] = lo
        else:
            if ni:
                a[i] = lo
            if nl:
                a[l] = hi
    return a


def _threshold(s):
    r, n = s.shape
    nchunks = n // 128
    if nchunks > _K and (nchunks & (nchunks - 1)) == 0:
        plan = _PLAN32 if nchunks == 32 else _bitonic_topk_plan(nchunks, _K)
        chunks = [s[:, g * 128:(g + 1) * 128] for g in range(nchunks)]
        cands = jnp.concatenate(_lane_topk(chunks, plan)[:_K], axis=1)
    else:
        cands = s
    m = jnp.max(cands, axis=1, keepdims=True)
    t = m
    c = jnp.sum(jnp.where(cands >= m, 1.0, 0.0), axis=1, keepdims=True)
    for _ in range(_K - 1):
        masked = jnp.where(cands < t, cands, -1.0)
        m = jnp.max(masked, axis=1, keepdims=True)
        cnt = jnp.sum(jnp.where(cands >= m, 1.0, 0.0), axis=1, keepdims=True)
        upd = c < float(_K)
        t = jnp.where(upd, m, t)
        c = jnp.where(upd, cnt, c)
    return t


def _phase1(wr_ref, wc_ref, s_out_ref, t_ref):
    i = pl.program_id(0)
    wr = wr_ref[...]
    wc = wc_ref[...]
    r, n = wr.shape
    s = jnp.maximum(jnp.maximum(wr, 0.0), jnp.maximum(wc, 0.0).T)
    col = jax.lax.broadcasted_iota(jnp.int32, (r, n), 1)
    row = jax.lax.broadcasted_iota(jnp.int32, (r, n), 0) + i * r
    s = jnp.where(col == row, s + 1.0, s)
    s_out_ref[...] = s
    t_ref[0, 0, :] = _threshold(s)[:, 0]


def _phase2(s_ref, t_ref, o_ref):
    s = s_ref[...]
    t = t_ref[0, 0, :][:, None]
    masked = jnp.where(s >= t, s, 0.0)
    ssum = jnp.sum(masked, axis=1, keepdims=True)
    o_ref[...] = masked * (1.0 / (ssum + 1e-8))


def kernel(W):
    n = W.shape[0]
    g = n // _R
    row_spec = pl.BlockSpec((_R, n), lambda i: (i, 0))
    col_spec = pl.BlockSpec((n, _R), lambda i: (0, i))
    vec_spec = pl.BlockSpec((1, 1, _R), lambda i: (i, 0, 0))

    s_full, t = pl.pallas_call(
        _phase1,
        grid=(g,),
        in_specs=[row_spec, col_spec],
        out_specs=[row_spec, vec_spec],
        out_shape=[
            jax.ShapeDtypeStruct((n, n), jnp.float32),
            jax.ShapeDtypeStruct((g, 1, _R), jnp.float32),
        ],
    )(W, W)

    out = pl.pallas_call(
        _phase2,
        grid=(g,),
        in_specs=[row_spec, vec_spec],
        out_specs=row_spec,
        out_shape=jax.ShapeDtypeStruct((n, n), jnp.float32),
    )(s_full, t)
    return out


# speedup vs baseline: 41.3257x; 1.3780x over previous
import jax
import jax.numpy as jnp
from jax.experimental import pallas as pl

_N = 4096
_R = 256
_K = 10


def _bitonic_topk_plan(n, k):
    ex = []
    kk = 2
    while kk <= n:
        j = kk // 2
        while j >= 1:
            for i in range(n):
                l = i ^ j
                if l > i:
                    ex.append((i, l, (i & kk) == 0))
            j //= 2
        kk *= 2
    needed = set(range(k))
    plan = []
    for (i, l, d) in reversed(ex):
        ni, nl = i in needed, l in needed
        if not (ni or nl):
            continue
        plan.append((i, l, d, ni, nl))
        needed.add(i)
        needed.add(l)
    plan.reverse()
    return plan


_PLAN32 = _bitonic_topk_plan(32, _K)


def _lane_topk(chunks, plan):
    a = list(chunks)
    for (i, l, desc, ni, nl) in plan:
        x, y = a[i], a[l]
        hi = jnp.maximum(x, y) if (ni if desc else nl) else None
        lo = jnp.minimum(x, y) if (nl if desc else ni) else None
        if desc:
            if ni:
                a[i] = hi
            if nl:
                a[l] = lo
        else:
            if ni:
                a[i] = lo
            if nl:
                a[l] = hi
    return a


def _threshold(s):
    r, n = s.shape
    nchunks = n // 128
    if nchunks > _K and (nchunks & (nchunks - 1)) == 0:
        plan = _PLAN32 if nchunks == 32 else _bitonic_topk_plan(nchunks, _K)
        chunks = [s[:, g * 128:(g + 1) * 128] for g in range(nchunks)]
        cands = jnp.concatenate(_lane_topk(chunks, plan)[:_K], axis=1)
    else:
        cands = s
    m = jnp.max(cands, axis=1, keepdims=True)
    t = m
    c = jnp.sum(jnp.where(cands >= m, 1.0, 0.0), axis=1, keepdims=True)
    for _ in range(_K - 1):
        masked = jnp.where(cands < t, cands, -1.0)
        m = jnp.max(masked, axis=1, keepdims=True)
        cnt = jnp.sum(jnp.where(cands >= m, 1.0, 0.0), axis=1, keepdims=True)
        upd = c < float(_K)
        t = jnp.where(upd, m, t)
        c = jnp.where(upd, cnt, c)
    return t


def _fused(wr_ref, wc_ref, o_ref):
    i = pl.program_id(0)
    wr = wr_ref[...]
    wc = wc_ref[...]
    r, n = wr.shape
    s = jnp.maximum(jnp.maximum(wr, 0.0), jnp.maximum(wc, 0.0).T)
    col = jax.lax.broadcasted_iota(jnp.int32, (r, n), 1)
    row = jax.lax.broadcasted_iota(jnp.int32, (r, n), 0) + i * r
    s = jnp.where(col == row, s + 1.0, s)
    t = _threshold(s)
    masked = jnp.where(s >= t, s, 0.0)
    ssum = jnp.sum(masked, axis=1, keepdims=True)
    o_ref[...] = masked * (1.0 / (ssum + 1e-8))


def kernel(W):
    n = W.shape[0]
    g = n // _R
    row_spec = pl.BlockSpec((_R, n), lambda i: (i, 0))
    col_spec = pl.BlockSpec((n, _R), lambda i: (0, i))
    return pl.pallas_call(
        _fused,
        grid=(g,),
        in_specs=[row_spec, col_spec],
        out_specs=row_spec,
        out_shape=jax.ShapeDtypeStruct((n, n), jnp.float32),
    )(W, W)
